# Initial kernel scaffold; baseline (speedup 1.0000x reference)
import jax
import jax.numpy as jnp
from jax.experimental import pallas as pl


def kernel(x_v, l_e_1, l_e_0, edge_index_1, edge_index_0, params):
    raise NotImplementedError("write your pallas kernel here")


# speedup vs baseline: 1.8754x; 1.8754x over previous
import functools

import jax
import jax.numpy as jnp
from jax import lax
from jax.experimental import pallas as pl
from jax.experimental.pallas import tpu as pltpu
from jax.experimental.pallas import tpu_sc as plsc

F32 = jnp.float32

N = 10000
NP = 10240
EMB = 128
H1 = 256
H2 = 128
T = 3
NE = 80000
EG = 2 * NE

NC = 2
NS = 16
NWORK = NC * NS
RPW = EG // NWORK
RPT = NE // NS
CH = 40
NCH = RPW // CH
NODE_BLK = NP // NS

@functools.cache
def _sc_mesh():
    return plsc.VectorSubcoreMesh(core_axis_name="c", subcore_axis_name="s",
                                  num_cores=NC, num_subcores=NS)


def _mm(a, b):
    return jnp.dot(a, b, preferred_element_type=F32)


def _node_init_body(x, w1, b1, w2, b2, w3, b3, wscat, wdcat,
                    h_out, tabs_out, tabd_out):
    h = jax.nn.relu(_mm(x[...], w1[...]) + b1[...])
    h = jax.nn.relu(_mm(h, w2[...]) + b2[...])
    h = _mm(h, w3[...]) + b3[...]
    h_out[...] = h
    ts = _mm(h, wscat[...])
    td = _mm(h, wdcat[...])
    tabs_out[0] = ts[:, :H1]
    tabs_out[1] = ts[:, H1:]
    tabd_out[0] = td[:, :H1]
    tabd_out[1] = td[:, H1:]


def _node_init(xp, pv, wscat, wdcat):
    nb = NP // NODE_BLK
    full = lambda s: pl.BlockSpec(s, lambda i: (0,) * len(s))
    return pl.pallas_call(
        _node_init_body,
        grid=(nb,),
        in_specs=[
            pl.BlockSpec((NODE_BLK, EMB), lambda i: (i, 0)),
            full((EMB, H1)), full((1, H1)),
            full((H1, H2)), full((1, H2)),
            full((H2, EMB)), full((1, EMB)),
            full((EMB, 2 * H1)), full((EMB, 2 * H1)),
        ],
        out_specs=[
            pl.BlockSpec((NODE_BLK, EMB), lambda i: (i, 0)),
            pl.BlockSpec((2, NODE_BLK, H1), lambda i: (0, i, 0)),
            pl.BlockSpec((2, NODE_BLK, H1), lambda i: (0, i, 0)),
        ],
        out_shape=[
            jax.ShapeDtypeStruct((NP, EMB), F32),
            jax.ShapeDtypeStruct((2, NP, H1), F32),
            jax.ShapeDtypeStruct((2, NP, H1), F32),
        ],
    )(xp, pv['W1'], pv['b1'][None], pv['W2'], pv['b2'][None],
      pv['W3'], pv['b3'][None], wscat, wdcat)


def _edge_feat_body(l, w1, b1, w2, b2, efold, ebias, e_out):
    h = jax.nn.relu(l[...] * w1[...] + b1[...])
    h = jax.nn.relu(_mm(h, w2[...]) + b2[...])
    e_out[...] = _mm(h, efold[0]) + ebias[0]


def _edge_feat(l_all, pe, efold_stack, ebias_stack):
    blk = 640
    nb = EG // blk
    half = nb // 2
    full = lambda s: pl.BlockSpec(s, lambda i: (0,) * len(s))
    return pl.pallas_call(
        _edge_feat_body,
        grid=(nb,),
        in_specs=[
            pl.BlockSpec((blk, 1), lambda i: (i, 0)),
            full((1, H1)), full((1, H1)),
            full((H1, H2)), full((1, H2)),
            pl.BlockSpec((1, H2, H1), lambda i: (i // half, 0, 0)),
            pl.BlockSpec((1, 1, H1), lambda i: (i // half, 0, 0)),
        ],
        out_specs=pl.BlockSpec((blk, H1), lambda i: (i, 0)),
        out_shape=jax.ShapeDtypeStruct((EG, H1), F32),
    )(l_all, pe['W1'], pe['b1'][None], pe['W2'], pe['b2'][None],
      efold_stack, ebias_stack)


def _edge_mlp_body(ga, gb, e, w2, b2, h2_out):
    h = jax.nn.relu(ga[...] + gb[...] + e[...])
    h2_out[...] = jax.nn.relu(_mm(h, w2[0]) + b2[0])


def _edge_mlp(ga, gb, e_all, w2_stack, b2_stack):
    blk = 640
    nb = EG // blk
    half = nb // 2
    return pl.pallas_call(
        _edge_mlp_body,
        grid=(nb,),
        in_specs=[
            pl.BlockSpec((blk, H1), lambda i: (i, 0)),
            pl.BlockSpec((blk, H1), lambda i: (i, 0)),
            pl.BlockSpec((blk, H1), lambda i: (i, 0)),
            pl.BlockSpec((1, H1, H2), lambda i: (i // half, 0, 0)),
            pl.BlockSpec((1, 1, H2), lambda i: (i // half, 0, 0)),
        ],
        out_specs=pl.BlockSpec((blk, H2), lambda i: (i, 0)),
        out_shape=jax.ShapeDtypeStruct((EG, H2), F32),
    )(ga, gb, e_all, w2_stack, b2_stack)


def _node_update_body(h, s, cnt, w1h, k1, k0, c1, c0, b1, w2, b2, w3, b3,
                      wscat, wdcat, h_out, tabs_out, tabd_out):
    hv = h[...]
    cnt1 = cnt[0][:, 0:1]
    cnt0 = cnt[1][:, 0:1]
    inv1 = 1.0 / jnp.maximum(cnt1, 1.0)
    inv0 = 1.0 / jnp.maximum(cnt0, 1.0)
    m1 = (cnt1 > 0.0).astype(F32)
    m0 = (cnt0 > 0.0).astype(F32)
    pre = (_mm(hv, w1h[...]) + _mm(s[0] * inv1, k1[...]) + m1 * c1[...]
           + _mm(s[1] * inv0, k0[...]) + m0 * c0[...] + b1[...])
    u = jax.nn.relu(pre)
    u = jax.nn.relu(_mm(u, w2[...]) + b2[...])
    hn = _mm(u, w3[...]) + b3[...] + hv
    h_out[...] = hn
    ts = _mm(hn, wscat[...])
    td = _mm(hn, wdcat[...])
    tabs_out[0] = ts[:, :H1]
    tabs_out[1] = ts[:, H1:]
    tabd_out[0] = td[:, :H1]
    tabd_out[1] = td[:, H1:]


def _node_update(h, s_part, cnt, wdict):
    nb = NP // NODE_BLK
    full = lambda s: pl.BlockSpec(s, lambda i: (0,) * len(s))
    return pl.pallas_call(
        _node_update_body,
        grid=(nb,),
        in_specs=[
            pl.BlockSpec((NODE_BLK, EMB), lambda i: (i, 0)),
            pl.BlockSpec((2, NODE_BLK, H2), lambda i: (0, i, 0)),
            pl.BlockSpec((2, NODE_BLK, 16), lambda i: (0, i, 0)),
            full((EMB, H1)), full((H2, H1)), full((H2, H1)),
            full((1, H1)), full((1, H1)), full((1, H1)),
            full((H1, H2)), full((1, H2)),
            full((H2, EMB)), full((1, EMB)),
            full((EMB, 2 * H1)), full((EMB, 2 * H1)),
        ],
        out_specs=[
            pl.BlockSpec((NODE_BLK, EMB), lambda i: (i, 0)),
            pl.BlockSpec((2, NODE_BLK, H1), lambda i: (0, i, 0)),
            pl.BlockSpec((2, NODE_BLK, H1), lambda i: (0, i, 0)),
        ],
        out_shape=[
            jax.ShapeDtypeStruct((NP, EMB), F32),
            jax.ShapeDtypeStruct((2, NP, H1), F32),
            jax.ShapeDtypeStruct((2, NP, H1), F32),
        ],
    )(h, s_part, cnt, wdict['w1h'], wdict['k1'], wdict['k0'],
      wdict['c1'], wdict['c0'], wdict['b1'], wdict['w2'], wdict['b2'],
      wdict['w3'], wdict['b3'], wdict['wscat'], wdict['wdcat'])


@functools.cache
def _sc_gather_kernel():
    return functools.partial(
        pl.kernel,
        out_type=(
            jax.ShapeDtypeStruct((EG, H1), F32),
            jax.ShapeDtypeStruct((EG, H1), F32),
        ),
        mesh=_sc_mesh(),
        scratch_types=[
            pltpu.VMEM((RPW,), jnp.int32),
            pltpu.VMEM((RPW,), jnp.int32),
            pltpu.VMEM((2, CH, H1), F32),
            pltpu.VMEM((2, CH, H1), F32),
            pltpu.SemaphoreType.DMA,
            pltpu.SemaphoreType.DMA,
            pltpu.SemaphoreType.DMA,
            pltpu.SemaphoreType.DMA,
        ],
    )(_sc_gather_body)


def _sc_gather_body(tab_s, tab_d, idx_a, idx_b, ga_out, gb_out,
                    ia_v, ib_v, buf_a, buf_b, sa0, sa1, sb0, sb1):
    wid = lax.axis_index("s") * NC + lax.axis_index("c")
    base = wid * RPW
    pltpu.sync_copy(idx_a.at[pl.ds(base, RPW)], ia_v)
    pltpu.sync_copy(idx_b.at[pl.ds(base, RPW)], ib_v)
    sems_a = (sa0, sa1)
    sems_b = (sb0, sb1)

    def start(c, b):
        off = c * CH
        pltpu.make_async_copy(
            tab_s.at[ia_v.at[pl.ds(off, CH)]], buf_a.at[b], sems_a[b]).start()
        pltpu.make_async_copy(
            tab_d.at[ib_v.at[pl.ds(off, CH)]], buf_b.at[b], sems_b[b]).start()

    def finish(c, b):
        off = c * CH
        pltpu.make_async_copy(
            tab_s.at[ia_v.at[pl.ds(off, CH)]], buf_a.at[b], sems_a[b]).wait()
        pltpu.make_async_copy(
            tab_d.at[ib_v.at[pl.ds(off, CH)]], buf_b.at[b], sems_b[b]).wait()
        pltpu.sync_copy(buf_a.at[b], ga_out.at[pl.ds(base + off, CH)])
        pltpu.sync_copy(buf_b.at[b], gb_out.at[pl.ds(base + off, CH)])

    start(0, 0)
    start(1, 1)

    def body(jj, carry):
        for b in range(2):
            c = jj * 2 + b

            @pl.when(c < NCH)
            def _():
                finish(c, b)

                @pl.when(c + 2 < NCH)
                def _():
                    start(c + 2, b)
        return carry

    lax.fori_loop(0, (NCH + 1) // 2, body, 0)


@functools.cache
def _sc_count_kernel():
    return functools.partial(
        pl.kernel,
        out_type=jax.ShapeDtypeStruct((2, NP, 16), F32),
        mesh=_sc_mesh(),
        scratch_types=[
            pltpu.VMEM((NCH, CH), jnp.int32),
            pltpu.VMEM((CH, 16), F32),
            pltpu.VMEM_SHARED((NP, 16), F32),
        ],
    )(_sc_count_body)


def _sc_count_body(dst_resh, zeros16, ones16, cnt_out, idx_v, ones_v, table):
    cid = lax.axis_index("c")
    sid = lax.axis_index("s")
    pltpu.sync_copy(zeros16, table.at[pl.ds(sid * NODE_BLK, NODE_BLK)])
    pltpu.sync_copy(dst_resh.at[cid, sid], idx_v)
    pltpu.sync_copy(ones16, ones_v)
    plsc.subcore_barrier()

    def body(j, carry):
        pltpu.sync_copy(ones_v, table.at[idx_v.at[j]], add=True)
        return carry

    lax.fori_loop(0, NCH, body, 0)
    plsc.subcore_barrier()
    pltpu.sync_copy(table.at[pl.ds(sid * NODE_BLK, NODE_BLK)],
                    cnt_out.at[cid, pl.ds(sid * NODE_BLK, NODE_BLK), :])


@functools.cache
def _sc_scatter_kernel():
    return functools.partial(
        pl.kernel,
        out_type=jax.ShapeDtypeStruct((2, NP, H2), F32),
        mesh=_sc_mesh(),
        scratch_types=[
            pltpu.VMEM((NCH, CH), jnp.int32),
            pltpu.VMEM((2, CH, H2), F32),
            pltpu.VMEM_SHARED((NP, H2), F32),
            pltpu.SemaphoreType.DMA,
            pltpu.SemaphoreType.DMA,
        ],
    )(_sc_scatter_body)


def _sc_scatter_body(h2, dst_resh, zeros128, s_out, idx_v, buf, table, s0, s1):
    cid = lax.axis_index("c")
    sid = lax.axis_index("s")
    rowbase = cid * NE + sid * RPT
    pltpu.sync_copy(zeros128, table.at[pl.ds(sid * NODE_BLK, NODE_BLK)])
    pltpu.sync_copy(dst_resh.at[cid, sid], idx_v)
    plsc.subcore_barrier()
    sems = (s0, s1)

    def start(c, b):
        pltpu.make_async_copy(
            h2.at[pl.ds(rowbase + c * CH, CH)], buf.at[b], sems[b]).start()

    def finish(c, b):
        pltpu.make_async_copy(
            h2.at[pl.ds(rowbase + c * CH, CH)], buf.at[b], sems[b]).wait()
        pltpu.sync_copy(buf.at[b], table.at[idx_v.at[c]], add=True)

    start(0, 0)
    start(1, 1)

    def body(jj, carry):
        for b in range(2):
            c = jj * 2 + b

            @pl.when(c < NCH)
            def _():
                finish(c, b)

                @pl.when(c + 2 < NCH)
                def _():
                    start(c + 2, b)
        return carry

    lax.fori_loop(0, (NCH + 1) // 2, body, 0)
    plsc.subcore_barrier()
    pltpu.sync_copy(table.at[pl.ds(sid * NODE_BLK, NODE_BLK)],
                    s_out.at[cid, pl.ds(sid * NODE_BLK, NODE_BLK), :])


def kernel(x_v, l_e_1, l_e_0, edge_index_1, edge_index_0, params):
    pv = params['mlp_v']
    pe = params['mlp_e']
    p1 = params['mlp_edge_1']
    p0 = params['mlp_edge_0']
    pa = params['mlp_aggr']

    xp = jnp.pad(x_v, ((0, NP - N), (0, 0)))
    src1 = edge_index_1[0].astype(jnp.int32)
    dst1 = edge_index_1[1].astype(jnp.int32)
    src0 = edge_index_0[0].astype(jnp.int32)
    dst0 = edge_index_0[1].astype(jnp.int32)
    idx_a = jnp.concatenate([src1, src0 + NP])
    idx_b = jnp.concatenate([dst1, dst0 + NP])
    dst_resh = jnp.stack([dst1, dst0]).reshape(2, NS, NCH, CH)
    l_all = jnp.concatenate([l_e_1, l_e_0], axis=0)

    w1s1, w1d1, w1c1 = p1['W1'][:EMB], p1['W1'][EMB:2 * EMB], p1['W1'][2 * EMB:]
    w1s0, w1d0, w1c0 = p0['W1'][:EMB], p0['W1'][EMB:2 * EMB], p0['W1'][2 * EMB:]
    wscat = jnp.concatenate([w1s1, w1s0], axis=1)
    wdcat = jnp.concatenate([w1d1, w1d0], axis=1)
    efold_stack = jnp.stack([pe['W3'] @ w1c1, pe['W3'] @ w1c0])
    ebias_stack = jnp.stack([(pe['b3'] @ w1c1 + p1['b1'])[None],
                             (pe['b3'] @ w1c0 + p0['b1'])[None]])
    w2_stack = jnp.stack([p1['W2'], p0['W2']])
    b2_stack = jnp.stack([p1['b2'][None], p0['b2'][None]])
    wa1 = pa['W1'][EMB:2 * EMB]
    wa0 = pa['W1'][2 * EMB:]
    upd = {
        'w1h': pa['W1'][:EMB],
        'k1': p1['W3'] @ wa1, 'k0': p0['W3'] @ wa0,
        'c1': (p1['b3'] @ wa1)[None], 'c0': (p0['b3'] @ wa0)[None],
        'b1': pa['b1'][None], 'w2': pa['W2'], 'b2': pa['b2'][None],
        'w3': pa['W3'], 'b3': pa['b3'][None],
        'wscat': wscat, 'wdcat': wdcat,
    }
    zeros16 = jnp.zeros((NODE_BLK, 16), F32)
    ones16 = jnp.ones((CH, 16), F32)
    zeros128 = jnp.zeros((NODE_BLK, H2), F32)

    h, tabs, tabd = _node_init(xp, pv, wscat, wdcat)
    e_all = _edge_feat(l_all, pe, efold_stack, ebias_stack)
    cnt = _sc_count_kernel()(dst_resh, zeros16, ones16)

    for _ in range(T):
        ga, gb = _sc_gather_kernel()(tabs.reshape(2 * NP, H1),
                                     tabd.reshape(2 * NP, H1), idx_a, idx_b)
        h2 = _edge_mlp(ga, gb, e_all, w2_stack, b2_stack)
        s_part = _sc_scatter_kernel()(h2, dst_resh, zeros128)
        h, tabs, tabd = _node_update(h, s_part, cnt, upd)

    return h[:N]


# speedup vs baseline: 2.3322x; 1.2436x over previous
import functools

import jax
import jax.numpy as jnp
from jax import lax
from jax.experimental import pallas as pl
from jax.experimental.pallas import tpu as pltpu
from jax.experimental.pallas import tpu_sc as plsc

F32 = jnp.float32
BF16 = jnp.bfloat16

N = 10000
NP = 10240
EMB = 128
H1 = 256
H2 = 128
T = 3
NE = 80000
EG = 2 * NE

NC = 2
NS = 16
NWORK = NC * NS
RPW = EG // NWORK
RPT = NE // NS
CH = 40
NCH = RPW // CH
NODE_BLK = NP // NS

@functools.cache
def _sc_mesh():
    return plsc.VectorSubcoreMesh(core_axis_name="c", subcore_axis_name="s",
                                  num_cores=NC, num_subcores=NS)


def _mm(a, b):
    return jnp.dot(a, b, preferred_element_type=F32)


def _node_init_body(x, w1, b1, w2, b2, w3, b3, wscat, wdcat,
                    h_out, tabs_out, tabd_out):
    h = jax.nn.relu(_mm(x[...], w1[...]) + b1[...])
    h = jax.nn.relu(_mm(h, w2[...]) + b2[...])
    h = _mm(h, w3[...]) + b3[...]
    h_out[...] = h
    _write_tables(h, wscat, wdcat, tabs_out, tabd_out)


def _pack2(a, b):
    lo = jax.lax.bitcast_convert_type(a.astype(BF16), jnp.uint16).astype(jnp.uint32)
    hi = jax.lax.bitcast_convert_type(b.astype(BF16), jnp.uint16).astype(jnp.uint32)
    return lo | (hi << 16)


def _unpack2(p):
    lo = jax.lax.bitcast_convert_type(
        (p & jnp.uint32(0xFFFF)).astype(jnp.uint16), BF16).astype(F32)
    hi = jax.lax.bitcast_convert_type(
        (p >> 16).astype(jnp.uint16), BF16).astype(F32)
    return lo, hi


def _write_tables(h, wscat, wdcat, tabs_out, tabd_out):
    ts = _mm(h, wscat[...])
    td = _mm(h, wdcat[...])
    tabs_out[0] = _pack2(ts[:, 0:128], ts[:, 128:256])
    tabs_out[1] = _pack2(ts[:, 256:384], ts[:, 384:512])
    tabd_out[0] = _pack2(td[:, 0:128], td[:, 128:256])
    tabd_out[1] = _pack2(td[:, 256:384], td[:, 384:512])


def _node_init(xp, pv, wscat, wdcat):
    nb = NP // NODE_BLK
    full = lambda s: pl.BlockSpec(s, lambda i: (0,) * len(s))
    return pl.pallas_call(
        _node_init_body,
        grid=(nb,),
        in_specs=[
            pl.BlockSpec((NODE_BLK, EMB), lambda i: (i, 0)),
            full((EMB, H1)), full((1, H1)),
            full((H1, H2)), full((1, H2)),
            full((H2, EMB)), full((1, EMB)),
            full((EMB, 2 * H1)), full((EMB, 2 * H1)),
        ],
        out_specs=[
            pl.BlockSpec((NODE_BLK, EMB), lambda i: (i, 0)),
            pl.BlockSpec((2, NODE_BLK, 128), lambda i: (0, i, 0)),
            pl.BlockSpec((2, NODE_BLK, 128), lambda i: (0, i, 0)),
        ],
        out_shape=[
            jax.ShapeDtypeStruct((NP, EMB), F32),
            jax.ShapeDtypeStruct((2, NP, 128), jnp.uint32),
            jax.ShapeDtypeStruct((2, NP, 128), jnp.uint32),
        ],
    )(xp, pv['W1'], pv['b1'][None], pv['W2'], pv['b2'][None],
      pv['W3'], pv['b3'][None], wscat, wdcat)


def _edge_feat_body(l, w1, b1, w2, b2, efold, ebias, e_out):
    h = jax.nn.relu(l[...] * w1[...] + b1[...])
    h = jax.nn.relu(_mm(h, w2[...]) + b2[...])
    e_out[...] = (_mm(h, efold[0]) + ebias[0]).astype(BF16)


def _edge_feat(l_all, pe, efold_stack, ebias_stack):
    blk = 640
    nb = EG // blk
    half = nb // 2
    full = lambda s: pl.BlockSpec(s, lambda i: (0,) * len(s))
    return pl.pallas_call(
        _edge_feat_body,
        grid=(nb,),
        in_specs=[
            pl.BlockSpec((blk, 1), lambda i: (i, 0)),
            full((1, H1)), full((1, H1)),
            full((H1, H2)), full((1, H2)),
            pl.BlockSpec((1, H2, H1), lambda i: (i // half, 0, 0)),
            pl.BlockSpec((1, 1, H1), lambda i: (i // half, 0, 0)),
        ],
        out_specs=pl.BlockSpec((blk, H1), lambda i: (i, 0)),
        out_shape=jax.ShapeDtypeStruct((EG, H1), BF16),
    )(l_all, pe['W1'], pe['b1'][None], pe['W2'], pe['b2'][None],
      efold_stack, ebias_stack)


def _edge_mlp_body(ga, gb, e, w2, b2, h2_out):
    gal, gah = _unpack2(ga[...])
    gbl, gbh = _unpack2(gb[...])
    ef = e[...].astype(F32)
    h1l = jax.nn.relu(gal + gbl + ef[:, :128])
    h1r = jax.nn.relu(gah + gbh + ef[:, 128:])
    h2_out[...] = jax.nn.relu(
        _mm(h1l, w2[0, :128]) + _mm(h1r, w2[0, 128:]) + b2[0])


def _edge_mlp(ga, gb, e_all, w2_stack, b2_stack):
    blk = 640
    nb = EG // blk
    half = nb // 2
    return pl.pallas_call(
        _edge_mlp_body,
        grid=(nb,),
        in_specs=[
            pl.BlockSpec((blk, 128), lambda i: (i, 0)),
            pl.BlockSpec((blk, 128), lambda i: (i, 0)),
            pl.BlockSpec((blk, H1), lambda i: (i, 0)),
            pl.BlockSpec((1, H1, H2), lambda i: (i // half, 0, 0)),
            pl.BlockSpec((1, 1, H2), lambda i: (i // half, 0, 0)),
        ],
        out_specs=pl.BlockSpec((blk, H2), lambda i: (i, 0)),
        out_shape=jax.ShapeDtypeStruct((EG, H2), F32),
    )(ga, gb, e_all, w2_stack, b2_stack)


def _node_update_body(h, s, cnt, w1h, k1, k0, c1, c0, b1, w2, b2, w3, b3,
                      wscat, wdcat, h_out, tabs_out, tabd_out):
    hv = h[...]
    cnt1 = cnt[0][:, 0:1]
    cnt0 = cnt[1][:, 0:1]
    inv1 = 1.0 / jnp.maximum(cnt1, 1.0)
    inv0 = 1.0 / jnp.maximum(cnt0, 1.0)
    m1 = (cnt1 > 0.0).astype(F32)
    m0 = (cnt0 > 0.0).astype(F32)
    pre = (_mm(hv, w1h[...]) + _mm(s[0] * inv1, k1[...]) + m1 * c1[...]
           + _mm(s[1] * inv0, k0[...]) + m0 * c0[...] + b1[...])
    u = jax.nn.relu(pre)
    u = jax.nn.relu(_mm(u, w2[...]) + b2[...])
    hn = _mm(u, w3[...]) + b3[...] + hv
    h_out[...] = hn
    _write_tables(hn, wscat, wdcat, tabs_out, tabd_out)


def _node_update(h, s_part, cnt, wdict):
    nb = NP // NODE_BLK
    full = lambda s: pl.BlockSpec(s, lambda i: (0,) * len(s))
    return pl.pallas_call(
        _node_update_body,
        grid=(nb,),
        in_specs=[
            pl.BlockSpec((NODE_BLK, EMB), lambda i: (i, 0)),
            pl.BlockSpec((2, NODE_BLK, H2), lambda i: (0, i, 0)),
            pl.BlockSpec((2, NODE_BLK, 16), lambda i: (0, i, 0)),
            full((EMB, H1)), full((H2, H1)), full((H2, H1)),
            full((1, H1)), full((1, H1)), full((1, H1)),
            full((H1, H2)), full((1, H2)),
            full((H2, EMB)), full((1, EMB)),
            full((EMB, 2 * H1)), full((EMB, 2 * H1)),
        ],
        out_specs=[
            pl.BlockSpec((NODE_BLK, EMB), lambda i: (i, 0)),
            pl.BlockSpec((2, NODE_BLK, 128), lambda i: (0, i, 0)),
            pl.BlockSpec((2, NODE_BLK, 128), lambda i: (0, i, 0)),
        ],
        out_shape=[
            jax.ShapeDtypeStruct((NP, EMB), F32),
            jax.ShapeDtypeStruct((2, NP, 128), jnp.uint32),
            jax.ShapeDtypeStruct((2, NP, 128), jnp.uint32),
        ],
    )(h, s_part, cnt, wdict['w1h'], wdict['k1'], wdict['k0'],
      wdict['c1'], wdict['c0'], wdict['b1'], wdict['w2'], wdict['b2'],
      wdict['w3'], wdict['b3'], wdict['wscat'], wdict['wdcat'])


@functools.cache
def _sc_gather_kernel():
    return functools.partial(
        pl.kernel,
        out_type=(
            jax.ShapeDtypeStruct((EG, 128), jnp.uint32),
            jax.ShapeDtypeStruct((EG, 128), jnp.uint32),
        ),
        mesh=_sc_mesh(),
        scratch_types=[
            pltpu.VMEM((RPW,), jnp.int32),
            pltpu.VMEM((RPW,), jnp.int32),
            pltpu.VMEM((2, CH, 128), jnp.uint32),
            pltpu.VMEM((2, CH, 128), jnp.uint32),
            pltpu.SemaphoreType.DMA,
            pltpu.SemaphoreType.DMA,
            pltpu.SemaphoreType.DMA,
            pltpu.SemaphoreType.DMA,
        ],
    )(_sc_gather_body)


def _sc_gather_body(tab_s, tab_d, idx_a, idx_b, ga_out, gb_out,
                    ia_v, ib_v, buf_a, buf_b, sa0, sa1, sb0, sb1):
    wid = lax.axis_index("s") * NC + lax.axis_index("c")
    base = wid * RPW
    pltpu.sync_copy(idx_a.at[pl.ds(base, RPW)], ia_v)
    pltpu.sync_copy(idx_b.at[pl.ds(base, RPW)], ib_v)
    sems_a = (sa0, sa1)
    sems_b = (sb0, sb1)

    def start(c, b):
        off = c * CH
        pltpu.make_async_copy(
            tab_s.at[ia_v.at[pl.ds(off, CH)]], buf_a.at[b], sems_a[b]).start()
        pltpu.make_async_copy(
            tab_d.at[ib_v.at[pl.ds(off, CH)]], buf_b.at[b], sems_b[b]).start()

    def finish(c, b):
        off = c * CH
        pltpu.make_async_copy(
            tab_s.at[ia_v.at[pl.ds(off, CH)]], buf_a.at[b], sems_a[b]).wait()
        pltpu.make_async_copy(
            tab_d.at[ib_v.at[pl.ds(off, CH)]], buf_b.at[b], sems_b[b]).wait()
        pltpu.sync_copy(buf_a.at[b], ga_out.at[pl.ds(base + off, CH)])
        pltpu.sync_copy(buf_b.at[b], gb_out.at[pl.ds(base + off, CH)])

    start(0, 0)
    start(1, 1)

    def body(jj, carry):
        for b in range(2):
            c = jj * 2 + b

            @pl.when(c < NCH)
            def _():
                finish(c, b)

                @pl.when(c + 2 < NCH)
                def _():
                    start(c + 2, b)
        return carry

    lax.fori_loop(0, (NCH + 1) // 2, body, 0)


@functools.cache
def _sc_count_kernel():
    return functools.partial(
        pl.kernel,
        out_type=jax.ShapeDtypeStruct((2, NP, 16), F32),
        mesh=_sc_mesh(),
        scratch_types=[
            pltpu.VMEM((NCH, CH), jnp.int32),
            pltpu.VMEM((CH, 16), F32),
            pltpu.VMEM_SHARED((NP, 16), F32),
        ],
    )(_sc_count_body)


def _sc_count_body(dst_resh, zeros16, ones16, cnt_out, idx_v, ones_v, table):
    cid = lax.axis_index("c")
    sid = lax.axis_index("s")
    pltpu.sync_copy(zeros16, table.at[pl.ds(sid * NODE_BLK, NODE_BLK)])
    pltpu.sync_copy(dst_resh.at[cid, sid], idx_v)
    pltpu.sync_copy(ones16, ones_v)
    plsc.subcore_barrier()

    def body(j, carry):
        pltpu.sync_copy(ones_v, table.at[idx_v.at[j]], add=True)
        return carry

    lax.fori_loop(0, NCH, body, 0)
    plsc.subcore_barrier()
    pltpu.sync_copy(table.at[pl.ds(sid * NODE_BLK, NODE_BLK)],
                    cnt_out.at[cid, pl.ds(sid * NODE_BLK, NODE_BLK), :])


@functools.cache
def _sc_scatter_kernel():
    return functools.partial(
        pl.kernel,
        out_type=jax.ShapeDtypeStruct((2, NP, H2), F32),
        mesh=_sc_mesh(),
        scratch_types=[
            pltpu.VMEM((NCH, CH), jnp.int32),
            pltpu.VMEM((2, CH, H2), F32),
            pltpu.VMEM_SHARED((NP, H2), F32),
            pltpu.SemaphoreType.DMA,
            pltpu.SemaphoreType.DMA,
        ],
    )(_sc_scatter_body)


def _sc_scatter_body(h2, dst_resh, zeros128, s_out, idx_v, buf, table, s0, s1):
    cid = lax.axis_index("c")
    sid = lax.axis_index("s")
    rowbase = cid * NE + sid * RPT
    pltpu.sync_copy(zeros128, table.at[pl.ds(sid * NODE_BLK, NODE_BLK)])
    pltpu.sync_copy(dst_resh.at[cid, sid], idx_v)
    plsc.subcore_barrier()
    sems = (s0, s1)

    def start(c, b):
        pltpu.make_async_copy(
            h2.at[pl.ds(rowbase + c * CH, CH)], buf.at[b], sems[b]).start()

    def finish(c, b):
        pltpu.make_async_copy(
            h2.at[pl.ds(rowbase + c * CH, CH)], buf.at[b], sems[b]).wait()
        pltpu.sync_copy(buf.at[b], table.at[idx_v.at[c]], add=True)

    start(0, 0)
    start(1, 1)

    def body(jj, carry):
        for b in range(2):
            c = jj * 2 + b

            @pl.when(c < NCH)
            def _():
                finish(c, b)

                @pl.when(c + 2 < NCH)
                def _():
                    start(c + 2, b)
        return carry

    lax.fori_loop(0, (NCH + 1) // 2, body, 0)
    plsc.subcore_barrier()
    pltpu.sync_copy(table.at[pl.ds(sid * NODE_BLK, NODE_BLK)],
                    s_out.at[cid, pl.ds(sid * NODE_BLK, NODE_BLK), :])


def kernel(x_v, l_e_1, l_e_0, edge_index_1, edge_index_0, params):
    pv = params['mlp_v']
    pe = params['mlp_e']
    p1 = params['mlp_edge_1']
    p0 = params['mlp_edge_0']
    pa = params['mlp_aggr']

    xp = jnp.pad(x_v, ((0, NP - N), (0, 0)))
    src1 = edge_index_1[0].astype(jnp.int32)
    dst1 = edge_index_1[1].astype(jnp.int32)
    src0 = edge_index_0[0].astype(jnp.int32)
    dst0 = edge_index_0[1].astype(jnp.int32)
    idx_a = jnp.concatenate([src1, src0 + NP])
    idx_b = jnp.concatenate([dst1, dst0 + NP])
    dst_resh = jnp.stack([dst1, dst0]).reshape(2, NS, NCH, CH)
    l_all = jnp.concatenate([l_e_1, l_e_0], axis=0)

    w1s1, w1d1, w1c1 = p1['W1'][:EMB], p1['W1'][EMB:2 * EMB], p1['W1'][2 * EMB:]
    w1s0, w1d0, w1c0 = p0['W1'][:EMB], p0['W1'][EMB:2 * EMB], p0['W1'][2 * EMB:]
    wscat = jnp.concatenate([w1s1, w1s0], axis=1)
    wdcat = jnp.concatenate([w1d1, w1d0], axis=1)
    efold_stack = jnp.stack([pe['W3'] @ w1c1, pe['W3'] @ w1c0])
    ebias_stack = jnp.stack([(pe['b3'] @ w1c1 + p1['b1'])[None],
                             (pe['b3'] @ w1c0 + p0['b1'])[None]])
    w2_stack = jnp.stack([p1['W2'], p0['W2']])
    b2_stack = jnp.stack([p1['b2'][None], p0['b2'][None]])
    wa1 = pa['W1'][EMB:2 * EMB]
    wa0 = pa['W1'][2 * EMB:]
    upd = {
        'w1h': pa['W1'][:EMB],
        'k1': p1['W3'] @ wa1, 'k0': p0['W3'] @ wa0,
        'c1': (p1['b3'] @ wa1)[None], 'c0': (p0['b3'] @ wa0)[None],
        'b1': pa['b1'][None], 'w2': pa['W2'], 'b2': pa['b2'][None],
        'w3': pa['W3'], 'b3': pa['b3'][None],
        'wscat': wscat, 'wdcat': wdcat,
    }
    zeros16 = jnp.zeros((NODE_BLK, 16), F32)
    ones16 = jnp.ones((CH, 16), F32)
    zeros128 = jnp.zeros((NODE_BLK, H2), F32)

    h, tabs, tabd = _node_init(xp, pv, wscat, wdcat)
    e_all = _edge_feat(l_all, pe, efold_stack, ebias_stack)
    cnt = _sc_count_kernel()(dst_resh, zeros16, ones16)

    for _ in range(T):
        ga, gb = _sc_gather_kernel()(tabs.reshape(2 * NP, 128),
                                     tabd.reshape(2 * NP, 128), idx_a, idx_b)
        h2 = _edge_mlp(ga, gb, e_all, w2_stack, b2_stack)
        s_part = _sc_scatter_kernel()(h2, dst_resh, zeros128)
        h, tabs, tabd = _node_update(h, s_part, cnt, upd)

    return h[:N]
